# baseline (speedup 1.0000x reference)
import jax
import jax.numpy as jnp
from jax import lax
from jax.experimental import pallas as pl
from jax.experimental.pallas import tpu as pltpu

E = 8
D = 1024
BT = 1024


def _moe_kernel(y_ref, wg_ref, wf_ref, be_ref, out_ref, z_ref, aux_ref):
    i = pl.program_id(0)

    y = y_ref[...]
    ybf = y.astype(jnp.bfloat16)

    logits = lax.dot_general(
        y, wg_ref[...], (((1,), (1,)), ((), ())),
        preferred_element_type=jnp.float32)
    m = jnp.max(logits, axis=-1, keepdims=True)
    ex = jnp.exp(logits - m)
    s = jnp.sum(ex, axis=-1, keepdims=True)
    p = ex / s
    pn = p / jnp.sum(p, axis=-1, keepdims=True)
    lse = m[:, 0] + jnp.log(s[:, 0])
    z_part = jnp.sum(lse * lse)
    p_part = jnp.sum(p)

    @pl.when(i == 0)
    def _init():
        z_ref[0, 0] = 0.0
        aux_ref[0, 0] = 0.0

    z_ref[0, 0] += z_part
    aux_ref[0, 0] += p_part

    acc0 = lax.dot_general(
        pn, be_ref[...], (((1,), (0,)), ((), ())),
        preferred_element_type=jnp.float32)
    acc1 = None
    for c in range(E // 2):
        z_c = lax.dot_general(
            ybf, wf_ref[2 * c * D:(2 * c + 2) * D, :], (((1,), (1,)), ((), ())),
            preferred_element_type=jnp.float32)
        e0, e1 = 2 * c, 2 * c + 1
        acc0 = acc0 + pn[:, e0:e0 + 1] * z_c[:, :D]
        term = pn[:, e1:e1 + 1] * z_c[:, D:]
        acc1 = term if acc1 is None else acc1 + term
    out_ref[...] = acc0 + acc1


@jax.jit
def kernel(x, W_gate, W_experts, b_experts):
    bs, seq, d = x.shape
    y = x.reshape(-1, d)
    T = y.shape[0]
    nt = T // BT

    Wflat = W_experts.reshape(E * D, D).astype(jnp.bfloat16)

    out, z, aux = pl.pallas_call(
        _moe_kernel,
        grid=(nt,),
        in_specs=[
            pl.BlockSpec((BT, D), lambda i: (i, 0)),
            pl.BlockSpec((E, D), lambda i: (0, 0)),
            pl.BlockSpec((E * D, D), lambda i: (0, 0)),
            pl.BlockSpec((E, D), lambda i: (0, 0)),
        ],
        out_specs=[
            pl.BlockSpec((BT, D), lambda i: (i, 0)),
            pl.BlockSpec(memory_space=pltpu.SMEM),
            pl.BlockSpec(memory_space=pltpu.SMEM),
        ],
        out_shape=[
            jax.ShapeDtypeStruct((T, D), jnp.float32),
            jax.ShapeDtypeStruct((1, 1), jnp.float32),
            jax.ShapeDtypeStruct((1, 1), jnp.float32),
        ],
    )(y, W_gate, Wflat, b_experts)

    z_loss = z[0, 0] / T
    aux_loss = aux[0, 0] * (E / T)
    return out.reshape(bs, seq, d), z_loss, aux_loss


# speedup vs baseline: 1.0879x; 1.0879x over previous
import jax
import jax.numpy as jnp
from jax import lax
from jax.experimental import pallas as pl
from jax.experimental.pallas import tpu as pltpu

E = 8
D = 1024
BT = 512


def _moe_kernel(y_ref, wg_ref, wf_ref, be_ref, out_ref, z_ref, aux_ref):
    i = pl.program_id(0)

    y = y_ref[...]
    ybf = y.astype(jnp.bfloat16)

    logits = lax.dot_general(
        y, wg_ref[...], (((1,), (1,)), ((), ())),
        preferred_element_type=jnp.float32)
    m = jnp.max(logits, axis=-1, keepdims=True)
    ex = jnp.exp(logits - m)
    s = jnp.sum(ex, axis=-1, keepdims=True)
    p = ex / s
    pn = p / jnp.sum(p, axis=-1, keepdims=True)
    lse = m[:, 0] + jnp.log(s[:, 0])
    z_part = jnp.sum(lse * lse)
    p_part = jnp.sum(p)

    @pl.when(i == 0)
    def _init():
        z_ref[0, 0] = 0.0
        aux_ref[0, 0] = 0.0

    z_ref[0, 0] += z_part
    aux_ref[0, 0] += p_part

    acc0 = lax.dot_general(
        pn, be_ref[...], (((1,), (0,)), ((), ())),
        preferred_element_type=jnp.float32)
    acc1 = None
    for e in range(E):
        z_e = lax.dot_general(
            ybf, wf_ref[e * D:(e + 1) * D, :].astype(jnp.bfloat16),
            (((1,), (1,)), ((), ())),
            preferred_element_type=jnp.float32)
        term = pn[:, e:e + 1] * z_e
        if e % 2 == 0:
            acc0 = acc0 + term
        else:
            acc1 = term if acc1 is None else acc1 + term
    out_ref[...] = acc0 + acc1


@jax.jit
def kernel(x, W_gate, W_experts, b_experts):
    bs, seq, d = x.shape
    y = x.reshape(-1, d)
    T = y.shape[0]
    nt = T // BT

    Wflat = W_experts.reshape(E * D, D)

    out, z, aux = pl.pallas_call(
        _moe_kernel,
        grid=(nt,),
        in_specs=[
            pl.BlockSpec((BT, D), lambda i: (i, 0)),
            pl.BlockSpec((E, D), lambda i: (0, 0)),
            pl.BlockSpec((E * D, D), lambda i: (0, 0)),
            pl.BlockSpec((E, D), lambda i: (0, 0)),
        ],
        out_specs=[
            pl.BlockSpec((BT, D), lambda i: (i, 0)),
            pl.BlockSpec(memory_space=pltpu.SMEM),
            pl.BlockSpec(memory_space=pltpu.SMEM),
        ],
        out_shape=[
            jax.ShapeDtypeStruct((T, D), jnp.float32),
            jax.ShapeDtypeStruct((1, 1), jnp.float32),
            jax.ShapeDtypeStruct((1, 1), jnp.float32),
        ],
    )(y, W_gate, Wflat, b_experts)

    z_loss = z[0, 0] / T
    aux_loss = aux[0, 0] * (E / T)
    return out.reshape(bs, seq, d), z_loss, aux_loss
